# Initial kernel scaffold; baseline (speedup 1.0000x reference)
import jax
import jax.numpy as jnp
from jax.experimental import pallas as pl


def kernel(keypoints3d, descriptors3d_db, descriptors3d_coarse_db, scores3d_db, params):
    raise NotImplementedError("write your pallas kernel here")


# speedup vs baseline: 3.4667x; 3.4667x over previous
import functools

import jax
import jax.numpy as jnp
from jax.experimental import pallas as pl


def _fps_kernel(npoint, n, n_pad, o_pad, xs_ref, ys_ref, zs_ref, out_ref):
    rn = n_pad // 128
    ro = o_pad // 128
    lin_n = (jax.lax.broadcasted_iota(jnp.int32, (rn, 128), 0) * 128
             + jax.lax.broadcasted_iota(jnp.int32, (rn, 128), 1))
    lin_o = (jax.lax.broadcasted_iota(jnp.int32, (ro, 128), 0) * 128
             + jax.lax.broadcasted_iota(jnp.int32, (ro, 128), 1))
    xs = xs_ref[...]
    ys = ys_ref[...]
    zs = zs_ref[...]
    dist0 = jnp.where(lin_n < n, jnp.float32(1e10), jnp.float32(-1.0))

    def body(i, carry):
        distance, farthest = carry
        out_ref[...] = jnp.where(lin_o == i, farthest, out_ref[...])
        sel = lin_n == farthest
        cx = jnp.sum(jnp.where(sel, xs, 0.0))
        cy = jnp.sum(jnp.where(sel, ys, 0.0))
        cz = jnp.sum(jnp.where(sel, zs, 0.0))
        dx = xs - cx
        dy = ys - cy
        dz = zs - cz
        d = dx * dx + dy * dy + dz * dz
        distance = jnp.minimum(distance, d)
        m = jnp.max(distance)
        farthest = jnp.min(jnp.where(distance == m, lin_n, n_pad)).astype(jnp.int32)
        return distance, farthest

    jax.lax.fori_loop(0, npoint, body, (dist0, jnp.int32(0)))


def _fps(xyz, npoint):
    n = xyz.shape[0]
    n_pad = -(-n // 1024) * 1024
    o_pad = -(-npoint // 1024) * 1024
    pad = n_pad - n
    cols = [jnp.pad(xyz[:, j], (0, pad)).reshape(n_pad // 128, 128) for j in range(3)]
    out = pl.pallas_call(
        functools.partial(_fps_kernel, npoint, n, n_pad, o_pad),
        out_shape=jax.ShapeDtypeStruct((o_pad // 128, 128), jnp.int32),
    )(*cols)
    return out.reshape(-1)[:npoint]


def _ball_query(radius, nsample, xyz, new_xyz):
    src = new_xyz[None]
    dst = xyz[None]
    sqrdists = (jnp.sum(src ** 2, -1)[:, :, None] + jnp.sum(dst ** 2, -1)[:, None, :]
                - 2.0 * jnp.einsum('bsc,bnc->bsn', src, dst))
    n = xyz.shape[0]
    s = new_xyz.shape[0]
    group_idx = jnp.broadcast_to(jnp.arange(n, dtype=jnp.int32), (1, s, n))
    group_idx = jnp.where(sqrdists > radius ** 2, n, group_idx)
    group_idx = jnp.sort(group_idx, axis=-1)[:, :, :nsample]
    group_first = group_idx[:, :, :1]
    group_idx = jnp.where(group_idx == n, group_first, group_idx)
    return group_idx[0]


def _dot_t(x, w):
    return jax.lax.dot_general(x, w, (((1,), (1,)), ((), ())),
                               preferred_element_type=jnp.float32)


def _accum_stats(y, step, s_ref, ss_ref):
    @pl.when(step == 0)
    def _():
        s_ref[...] = jnp.zeros_like(s_ref)
        ss_ref[...] = jnp.zeros_like(ss_ref)

    s_ref[...] += jnp.sum(y, axis=0, keepdims=True)
    ss_ref[...] += jnp.sum(y * y, axis=0, keepdims=True)


def _layer_first_kernel(xg_ref, pg_ref, wx_ref, wp_ref, b_ref, y_ref, s_ref, ss_ref):
    y = (_dot_t(pg_ref[...], wp_ref[...]) + _dot_t(xg_ref[...], wx_ref[...])
         + b_ref[...])
    y_ref[...] = y
    _accum_stats(y, pl.program_id(0), s_ref, ss_ref)


def _layer_mid_kernel(x_ref, w_ref, b_ref, a_ref, c_ref, y_ref, s_ref, ss_ref):
    x = jnp.maximum(x_ref[...] * a_ref[...] + c_ref[...], 0.0)
    y = _dot_t(x, w_ref[...]) + b_ref[...]
    y_ref[...] = y
    _accum_stats(y, pl.program_id(0), s_ref, ss_ref)


def _layer_last_kernel(gtile, k, x_ref, w_ref, b_ref, a_ref, c_ref,
                       mx_ref, mn_ref, s_ref, ss_ref):
    x = jnp.maximum(x_ref[...] * a_ref[...] + c_ref[...], 0.0)
    y = _dot_t(x, w_ref[...]) + b_ref[...]
    cout = y.shape[-1]
    g = y.reshape(gtile, k, cout)
    mx_ref[...] = jnp.max(g, axis=1)
    mn_ref[...] = jnp.min(g, axis=1)
    _accum_stats(y, pl.program_id(0), s_ref, ss_ref)


def _bn_affine(s, ss, count, layer):
    mean = s[0] / count
    var = ss[0] / count - mean * mean
    a = layer['g'] / jnp.sqrt(var + 1e-5)
    c = layer['beta'] - mean * a
    return a[None], c[None]


def _rep(shape):
    return pl.BlockSpec(shape, lambda i: (0, 0))


def _mlp(xg, pg, layers, s_groups, k):
    p = s_groups * k
    cp = pg.shape[1]
    c1 = layers[0]['W'].shape[0]
    c2 = layers[1]['W'].shape[0]
    c3 = layers[2]['W'].shape[0]
    tp = 2048
    gtile = 8
    tpl = gtile * k
    f32 = jnp.float32

    wx = layers[0]['W'][:, :3]
    wp = layers[0]['W'][:, 3:]
    y1, s1, ss1 = pl.pallas_call(
        _layer_first_kernel,
        grid=(p // tp,),
        in_specs=[pl.BlockSpec((tp, 3), lambda i: (i, 0)),
                  pl.BlockSpec((tp, cp), lambda i: (i, 0)),
                  _rep((c1, 3)), _rep((c1, cp)), _rep((1, c1))],
        out_specs=[pl.BlockSpec((tp, c1), lambda i: (i, 0)),
                   _rep((1, c1)), _rep((1, c1))],
        out_shape=[jax.ShapeDtypeStruct((p, c1), f32),
                   jax.ShapeDtypeStruct((1, c1), f32),
                   jax.ShapeDtypeStruct((1, c1), f32)],
    )(xg, pg, wx, wp, layers[0]['b'][None])
    a1, c1aff = _bn_affine(s1, ss1, p, layers[0])

    y2, s2, ss2 = pl.pallas_call(
        _layer_mid_kernel,
        grid=(p // tp,),
        in_specs=[pl.BlockSpec((tp, c1), lambda i: (i, 0)),
                  _rep((c2, c1)), _rep((1, c2)), _rep((1, c1)), _rep((1, c1))],
        out_specs=[pl.BlockSpec((tp, c2), lambda i: (i, 0)),
                   _rep((1, c2)), _rep((1, c2))],
        out_shape=[jax.ShapeDtypeStruct((p, c2), f32),
                   jax.ShapeDtypeStruct((1, c2), f32),
                   jax.ShapeDtypeStruct((1, c2), f32)],
    )(y1, layers[1]['W'], layers[1]['b'][None], a1, c1aff)
    a2, c2aff = _bn_affine(s2, ss2, p, layers[1])

    mx, mn, s3, ss3 = pl.pallas_call(
        functools.partial(_layer_last_kernel, gtile, k),
        grid=(s_groups // gtile,),
        in_specs=[pl.BlockSpec((tpl, c2), lambda i: (i, 0)),
                  _rep((c3, c2)), _rep((1, c3)), _rep((1, c2)), _rep((1, c2))],
        out_specs=[pl.BlockSpec((gtile, c3), lambda i: (i, 0)),
                   pl.BlockSpec((gtile, c3), lambda i: (i, 0)),
                   _rep((1, c3)), _rep((1, c3))],
        out_shape=[jax.ShapeDtypeStruct((s_groups, c3), f32),
                   jax.ShapeDtypeStruct((s_groups, c3), f32),
                   jax.ShapeDtypeStruct((1, c3), f32),
                   jax.ShapeDtypeStruct((1, c3), f32)],
    )(y2, layers[2]['W'], layers[2]['b'][None], a2, c2aff)
    a3, c3aff = _bn_affine(s3, ss3, p, layers[2])

    pooled = jnp.where(a3 > 0,
                       jnp.maximum(mx * a3 + c3aff, 0.0),
                       jnp.maximum(mn * a3 + c3aff, 0.0))
    return pooled


_NPOINT1, _RADIUS1, _NSAMPLE1 = 5000, 0.2, 256
_NPOINT2, _RADIUS2, _NSAMPLE2 = 256, 0.4, 128


def kernel(keypoints3d, descriptors3d_db, descriptors3d_coarse_db, scores3d_db, params):
    xyz1 = keypoints3d[0]
    pts1 = jnp.transpose(descriptors3d_db[0])

    fps1 = _fps(xyz1, _NPOINT1)
    new_xyz1 = xyz1[fps1]
    idx1 = _ball_query(_RADIUS1, _NSAMPLE1, xyz1, new_xyz1)
    gx1 = (xyz1[idx1] - new_xyz1[:, None, :]).reshape(-1, 3)
    gp1 = pts1[idx1].reshape(-1, pts1.shape[1])
    l1_points = _mlp(gx1, gp1, params['sa1'], _NPOINT1, _NSAMPLE1)

    fps2 = _fps(new_xyz1, _NPOINT2)
    new_xyz2 = new_xyz1[fps2]
    idx2 = _ball_query(_RADIUS2, _NSAMPLE2, new_xyz1, new_xyz2)
    gx2 = (new_xyz1[idx2] - new_xyz2[:, None, :]).reshape(-1, 3)
    gp2 = l1_points[idx2].reshape(-1, l1_points.shape[1])
    l2_pooled = _mlp(gx2, gp2, params['sa2'], _NPOINT2, _NSAMPLE2)

    fps_idx = fps1[fps2][None]
    keypoints3d_new = new_xyz2[None]
    l2_points = jnp.transpose(l2_pooled)[None]
    new_desc_coarse = descriptors3d_coarse_db[:, :, fps_idx[0]]
    new_scores = scores3d_db[0][fps_idx[0]][None]
    return keypoints3d_new, l2_points, new_desc_coarse, new_scores, fps_idx


# speedup vs baseline: 6.4259x; 1.8536x over previous
import functools

import jax
import jax.numpy as jnp
from jax.experimental import pallas as pl


def _fps_kernel(npoint, n, n_pad, o_pad, xs_ref, ys_ref, zs_ref, out_ref):
    rn = n_pad // 128
    ro = o_pad // 128
    lin_n = (jax.lax.broadcasted_iota(jnp.int32, (rn, 128), 0) * 128
             + jax.lax.broadcasted_iota(jnp.int32, (rn, 128), 1))
    lin_o = (jax.lax.broadcasted_iota(jnp.int32, (ro, 128), 0) * 128
             + jax.lax.broadcasted_iota(jnp.int32, (ro, 128), 1))
    xs = xs_ref[...]
    ys = ys_ref[...]
    zs = zs_ref[...]
    dist0 = jnp.where(lin_n < n, jnp.float32(1e10), jnp.float32(-1.0))

    def body(i, carry):
        distance, farthest = carry
        out_ref[...] = jnp.where(lin_o == i, farthest, out_ref[...])
        sel = lin_n == farthest
        cx = jnp.sum(jnp.where(sel, xs, 0.0))
        cy = jnp.sum(jnp.where(sel, ys, 0.0))
        cz = jnp.sum(jnp.where(sel, zs, 0.0))
        dx = xs - cx
        dy = ys - cy
        dz = zs - cz
        d = dx * dx + dy * dy + dz * dz
        distance = jnp.minimum(distance, d)
        m = jnp.max(distance)
        farthest = jnp.min(jnp.where(distance == m, lin_n, n_pad)).astype(jnp.int32)
        return distance, farthest

    jax.lax.fori_loop(0, npoint, body, (dist0, jnp.int32(0)))


def _fps(xyz, npoint):
    n = xyz.shape[0]
    n_pad = -(-n // 1024) * 1024
    o_pad = -(-npoint // 1024) * 1024
    pad = n_pad - n
    cols = [jnp.pad(xyz[:, j], (0, pad)).reshape(n_pad // 128, 128) for j in range(3)]
    out = pl.pallas_call(
        functools.partial(_fps_kernel, npoint, n, n_pad, o_pad),
        out_shape=jax.ShapeDtypeStruct((o_pad // 128, 128), jnp.int32),
    )(*cols)
    return out.reshape(-1)[:npoint]


def _ball_query(radius, nsample, xyz, new_xyz):
    src = new_xyz[None]
    dst = xyz[None]
    sqrdists = (jnp.sum(src ** 2, -1)[:, :, None] + jnp.sum(dst ** 2, -1)[:, None, :]
                - 2.0 * jnp.einsum('bsc,bnc->bsn', src, dst))
    n = xyz.shape[0]
    s = new_xyz.shape[0]
    mask = jnp.logical_not(sqrdists[0] > radius ** 2)
    c = jnp.cumsum(mask.astype(jnp.int32), axis=-1)
    targets = jnp.arange(1, nsample + 1, dtype=jnp.int32)
    idx = jax.vmap(lambda cr: jnp.searchsorted(cr, targets, side='left'))(c)
    idx = idx.astype(jnp.int32)
    count = c[:, -1:]
    j = jnp.arange(nsample, dtype=jnp.int32)[None]
    return jnp.where(j < count, idx, idx[:, :1])


def _dot_t(x, w):
    return jax.lax.dot_general(x, w, (((1,), (1,)), ((), ())),
                               preferred_element_type=jnp.float32)


def _accum_stats(y, step, s_ref, ss_ref):
    @pl.when(step == 0)
    def _():
        s_ref[...] = jnp.zeros_like(s_ref)
        ss_ref[...] = jnp.zeros_like(ss_ref)

    s_ref[...] += jnp.sum(y, axis=0, keepdims=True)
    ss_ref[...] += jnp.sum(y * y, axis=0, keepdims=True)


def _layer_first_kernel(xg_ref, pg_ref, wx_ref, wp_ref, b_ref, y_ref, s_ref, ss_ref):
    y = (_dot_t(pg_ref[...], wp_ref[...]) + _dot_t(xg_ref[...], wx_ref[...])
         + b_ref[...])
    y_ref[...] = y
    _accum_stats(y, pl.program_id(0), s_ref, ss_ref)


def _layer_mid_kernel(x_ref, w_ref, b_ref, a_ref, c_ref, y_ref, s_ref, ss_ref):
    x = jnp.maximum(x_ref[...] * a_ref[...] + c_ref[...], 0.0)
    y = _dot_t(x, w_ref[...]) + b_ref[...]
    y_ref[...] = y
    _accum_stats(y, pl.program_id(0), s_ref, ss_ref)


def _layer_last_kernel(gtile, k, x_ref, w_ref, b_ref, a_ref, c_ref,
                       mx_ref, mn_ref, s_ref, ss_ref):
    x = jnp.maximum(x_ref[...] * a_ref[...] + c_ref[...], 0.0)
    y = _dot_t(x, w_ref[...]) + b_ref[...]
    cout = y.shape[-1]
    g = y.reshape(gtile, k, cout)
    mx_ref[...] = jnp.max(g, axis=1)
    mn_ref[...] = jnp.min(g, axis=1)
    _accum_stats(y, pl.program_id(0), s_ref, ss_ref)


def _bn_affine(s, ss, count, layer):
    mean = s[0] / count
    var = ss[0] / count - mean * mean
    a = layer['g'] / jnp.sqrt(var + 1e-5)
    c = layer['beta'] - mean * a
    return a[None], c[None]


def _rep(shape):
    return pl.BlockSpec(shape, lambda i: (0, 0))


def _mlp(xg, pg, layers, s_groups, k):
    p = s_groups * k
    cp = pg.shape[1]
    c1 = layers[0]['W'].shape[0]
    c2 = layers[1]['W'].shape[0]
    c3 = layers[2]['W'].shape[0]
    tp = 2048
    gtile = 8
    tpl = gtile * k
    f32 = jnp.float32

    wx = layers[0]['W'][:, :3]
    wp = layers[0]['W'][:, 3:]
    y1, s1, ss1 = pl.pallas_call(
        _layer_first_kernel,
        grid=(p // tp,),
        in_specs=[pl.BlockSpec((tp, 3), lambda i: (i, 0)),
                  pl.BlockSpec((tp, cp), lambda i: (i, 0)),
                  _rep((c1, 3)), _rep((c1, cp)), _rep((1, c1))],
        out_specs=[pl.BlockSpec((tp, c1), lambda i: (i, 0)),
                   _rep((1, c1)), _rep((1, c1))],
        out_shape=[jax.ShapeDtypeStruct((p, c1), f32),
                   jax.ShapeDtypeStruct((1, c1), f32),
                   jax.ShapeDtypeStruct((1, c1), f32)],
    )(xg, pg, wx, wp, layers[0]['b'][None])
    a1, c1aff = _bn_affine(s1, ss1, p, layers[0])

    y2, s2, ss2 = pl.pallas_call(
        _layer_mid_kernel,
        grid=(p // tp,),
        in_specs=[pl.BlockSpec((tp, c1), lambda i: (i, 0)),
                  _rep((c2, c1)), _rep((1, c2)), _rep((1, c1)), _rep((1, c1))],
        out_specs=[pl.BlockSpec((tp, c2), lambda i: (i, 0)),
                   _rep((1, c2)), _rep((1, c2))],
        out_shape=[jax.ShapeDtypeStruct((p, c2), f32),
                   jax.ShapeDtypeStruct((1, c2), f32),
                   jax.ShapeDtypeStruct((1, c2), f32)],
    )(y1, layers[1]['W'], layers[1]['b'][None], a1, c1aff)
    a2, c2aff = _bn_affine(s2, ss2, p, layers[1])

    mx, mn, s3, ss3 = pl.pallas_call(
        functools.partial(_layer_last_kernel, gtile, k),
        grid=(s_groups // gtile,),
        in_specs=[pl.BlockSpec((tpl, c2), lambda i: (i, 0)),
                  _rep((c3, c2)), _rep((1, c3)), _rep((1, c2)), _rep((1, c2))],
        out_specs=[pl.BlockSpec((gtile, c3), lambda i: (i, 0)),
                   pl.BlockSpec((gtile, c3), lambda i: (i, 0)),
                   _rep((1, c3)), _rep((1, c3))],
        out_shape=[jax.ShapeDtypeStruct((s_groups, c3), f32),
                   jax.ShapeDtypeStruct((s_groups, c3), f32),
                   jax.ShapeDtypeStruct((1, c3), f32),
                   jax.ShapeDtypeStruct((1, c3), f32)],
    )(y2, layers[2]['W'], layers[2]['b'][None], a2, c2aff)
    a3, c3aff = _bn_affine(s3, ss3, p, layers[2])

    pooled = jnp.where(a3 > 0,
                       jnp.maximum(mx * a3 + c3aff, 0.0),
                       jnp.maximum(mn * a3 + c3aff, 0.0))
    return pooled


_NPOINT1, _RADIUS1, _NSAMPLE1 = 5000, 0.2, 256
_NPOINT2, _RADIUS2, _NSAMPLE2 = 256, 0.4, 128


def kernel(keypoints3d, descriptors3d_db, descriptors3d_coarse_db, scores3d_db, params):
    xyz1 = keypoints3d[0]
    pts1 = jnp.transpose(descriptors3d_db[0])

    fps1 = _fps(xyz1, _NPOINT1)
    new_xyz1 = xyz1[fps1]
    idx1 = _ball_query(_RADIUS1, _NSAMPLE1, xyz1, new_xyz1)
    gx1 = (xyz1[idx1] - new_xyz1[:, None, :]).reshape(-1, 3)
    gp1 = pts1[idx1].reshape(-1, pts1.shape[1])
    l1_points = _mlp(gx1, gp1, params['sa1'], _NPOINT1, _NSAMPLE1)

    fps2 = _fps(new_xyz1, _NPOINT2)
    new_xyz2 = new_xyz1[fps2]
    idx2 = _ball_query(_RADIUS2, _NSAMPLE2, new_xyz1, new_xyz2)
    gx2 = (new_xyz1[idx2] - new_xyz2[:, None, :]).reshape(-1, 3)
    gp2 = l1_points[idx2].reshape(-1, l1_points.shape[1])
    l2_pooled = _mlp(gx2, gp2, params['sa2'], _NPOINT2, _NSAMPLE2)

    fps_idx = fps1[fps2][None]
    keypoints3d_new = new_xyz2[None]
    l2_points = jnp.transpose(l2_pooled)[None]
    new_desc_coarse = descriptors3d_coarse_db[:, :, fps_idx[0]]
    new_scores = scores3d_db[0][fps_idx[0]][None]
    return keypoints3d_new, l2_points, new_desc_coarse, new_scores, fps_idx
